# baseline (speedup 1.0000x reference)
import functools

import jax
import jax.numpy as jnp
from jax import lax
from jax.experimental import pallas as pl
from jax.experimental.pallas import tpu as pltpu
from jax.experimental.pallas import tpu_sc as plsc

_T = 2048
_D = 768
_E = 8
_H = 3072
_CAP = 512
_TRASH = _E * _CAP
_XG_ROWS = _TRASH + 8
_NW = 32
_CHUNK = (2 * _T) // _NW
_NEG_INF = float("-inf")


def _router_body(xf_ref, nf_ref, w_ref, b_ref,
                 slot_s_ref, slot_g_ref, g1_ref, g2_ref, ns_ref):
    r = jnp.dot(xf_ref[...].astype(jnp.bfloat16),
                w_ref[...].astype(jnp.bfloat16),
                preferred_element_type=jnp.float32) + b_ref[...]
    logits = r[:, 0:_E]
    nlog = r[:, _E:2 * _E]
    z = r[:, 2 * _E:2 * _E + 1]
    sp = jnp.maximum(nlog, 0.0) + jnp.log1p(jnp.exp(-jnp.abs(nlog)))
    noisy = logits + nf_ref[...] * sp

    iota8 = lax.broadcasted_iota(jnp.int32, (_T, _E), 1)
    v1 = jnp.max(noisy, axis=1, keepdims=True)
    i1 = jnp.min(jnp.where(noisy == v1, iota8, _E), axis=1, keepdims=True)
    oh1 = iota8 == i1
    noisy2 = jnp.where(oh1, _NEG_INF, noisy)
    v2 = jnp.max(noisy2, axis=1, keepdims=True)
    i2 = jnp.min(jnp.where(noisy2 == v2, iota8, _E), axis=1, keepdims=True)
    oh2 = iota8 == i2

    ed = jnp.exp(v2 - v1)
    g1 = 1.0 / (1.0 + ed)
    g2 = ed / (1.0 + ed)

    ns = z <= 0.0
    nsf = ns.astype(jnp.float32)

    mask = jnp.where(jnp.logical_and(jnp.logical_or(oh1, oh2), ns), 1.0, 0.0)
    cum = mask
    sh = 1
    while sh < _T:
        cum = cum + jnp.concatenate(
            [jnp.zeros((sh, _E), jnp.float32), cum[:_T - sh]], axis=0)
        sh *= 2
    rank = cum - mask

    num_ns = jnp.sum(nsf, axis=0, keepdims=True)
    cap = jnp.floor(num_ns * 0.25)

    rank1 = jnp.sum(jnp.where(oh1, rank, 0.0), axis=1, keepdims=True)
    rank2 = jnp.sum(jnp.where(oh2, rank, 0.0), axis=1, keepdims=True)
    val1 = jnp.logical_and(ns, rank1 < cap)
    val2 = jnp.logical_and(ns, rank2 < cap)

    slot1 = i1 * _CAP + rank1.astype(jnp.int32)
    slot2 = i2 * _CAP + rank2.astype(jnp.int32)
    slot_s_ref[...] = jnp.concatenate(
        [jnp.where(val1, slot1, _TRASH), jnp.where(val2, slot2, _TRASH)], axis=1)
    slot_g_ref[...] = jnp.concatenate(
        [jnp.where(val1, slot1, 0), jnp.where(val2, slot2, 0)], axis=1)
    g1_ref[...] = g1 * val1.astype(jnp.float32)
    g2_ref[...] = g2 * val2.astype(jnp.float32)
    ns_ref[...] = nsf


def _router(xf, nf, wcat, bcat):
    return pl.pallas_call(
        _router_body,
        out_shape=[
            jax.ShapeDtypeStruct((_T, 2), jnp.int32),
            jax.ShapeDtypeStruct((_T, 2), jnp.int32),
            jax.ShapeDtypeStruct((_T, 1), jnp.float32),
            jax.ShapeDtypeStruct((_T, 1), jnp.float32),
            jax.ShapeDtypeStruct((_T, 1), jnp.float32),
        ],
    )(xf, nf, wcat, bcat)


_NSTR = 4
_SUB = _CHUNK // _NSTR


def _dispatch(xf, idx):
    @functools.partial(
        pl.kernel,
        out_type=jax.ShapeDtypeStruct((_XG_ROWS, _D), jnp.float32),
        mesh=plsc.VectorSubcoreMesh(core_axis_name="c", subcore_axis_name="s"),
        scratch_types=(
            [pltpu.VMEM((_SUB,), jnp.int32) for _ in range(_NSTR)]
            + [pltpu.VMEM((_CHUNK, _D), jnp.float32)]
            + [pltpu.SemaphoreType.DMA for _ in range(_NSTR + 1)]
        ),
    )
    def k(xf_hbm, idx_hbm, xg_hbm, i0, i1, i2, i3, rows_v, s0, s1, s2, s3, sr):
        wid = lax.axis_index("s") * 2 + lax.axis_index("c")
        base = wid * _CHUNK
        tok = lax.rem(base, _T)
        rcp = pltpu.make_async_copy(xf_hbm.at[pl.ds(tok, _CHUNK)], rows_v, sr)
        rcp.start()
        ivs = [i0, i1, i2, i3]
        sems = [s0, s1, s2, s3]
        for c in range(_NSTR):
            pltpu.sync_copy(idx_hbm.at[pl.ds(base + c * _SUB, _SUB)], ivs[c])
        rcp.wait()
        cps = []
        for c in range(_NSTR):
            cp = pltpu.make_async_copy(
                rows_v.at[pl.ds(c * _SUB, _SUB)], xg_hbm.at[ivs[c]], sems[c])
            cp.start()
            cps.append(cp)
        for cp in cps:
            cp.wait()

    return k(xf, idx)


def _mlp_body(xg_ref, w1_ref, b1_ref, w2_ref, b2_ref, y_ref, acc_ref):
    h_idx = pl.program_id(1)
    xraw = xg_ref[...]
    xb = jnp.where(jnp.abs(xraw) < 1e30, xraw, 0.0).astype(jnp.bfloat16)
    w1 = w1_ref[0]
    h = jnp.dot(xb, w1, preferred_element_type=jnp.float32) + b1_ref[0]
    hb = jnp.maximum(h, 0.0).astype(jnp.bfloat16)
    w2 = w2_ref[0]
    part = jnp.dot(hb, w2, preferred_element_type=jnp.float32)

    @pl.when(h_idx == 0)
    def _():
        acc_ref[...] = part + b2_ref[0]

    @pl.when(h_idx != 0)
    def _():
        y_ref[...] = (acc_ref[...] + part).astype(jnp.bfloat16)


_HB = 2
_HBLK = _H // _HB


def _mlp(xg, W1, b1, W2, b2):
    return pl.pallas_call(
        _mlp_body,
        grid=(_E, _HB),
        in_specs=[
            pl.BlockSpec((_CAP, _D), lambda e, h: (e, 0)),
            pl.BlockSpec((1, _D, _HBLK), lambda e, h: (e, 0, h)),
            pl.BlockSpec((1, 1, _HBLK), lambda e, h: (e, 0, h)),
            pl.BlockSpec((1, _HBLK, _D), lambda e, h: (e, h, 0)),
            pl.BlockSpec((1, 1, _D), lambda e, h: (e, 0, 0)),
        ],
        out_specs=pl.BlockSpec((_CAP, _D), lambda e, h: (e, 0)),
        out_shape=jax.ShapeDtypeStruct((_E * _CAP, _D), jnp.bfloat16),
        scratch_shapes=[pltpu.VMEM((_CAP, _D), jnp.float32)],
    )(xg, W1, b1.reshape(_E, 1, _H), W2, b2.reshape(_E, 1, _D))


_CB = 4


def _combine_body(sg_ref, g1_ref, g2_ref, ns_ref, xf_ref, y_ref, out_ref):
    s1 = sg_ref[...][:, 0:1]
    s2 = sg_ref[...][:, 1:2]
    iota = lax.broadcasted_iota(jnp.int32, (_T // _CB, _TRASH), 1)
    c = (jnp.where(iota == s1, g1_ref[...], 0.0)
         + jnp.where(iota == s2, g2_ref[...], 0.0)).astype(jnp.bfloat16)
    upd = jnp.dot(c, y_ref[...], preferred_element_type=jnp.float32)
    out_ref[...] = jnp.where(ns_ref[...] > 0.0, upd, xf_ref[...])


def _combine(slot_g, g1, g2, nsf, xf, y):
    blk = _T // _CB
    return pl.pallas_call(
        _combine_body,
        grid=(_CB,),
        in_specs=[
            pl.BlockSpec((blk, 2), lambda i: (i, 0)),
            pl.BlockSpec((blk, 1), lambda i: (i, 0)),
            pl.BlockSpec((blk, 1), lambda i: (i, 0)),
            pl.BlockSpec((blk, 1), lambda i: (i, 0)),
            pl.BlockSpec((blk, _D), lambda i: (i, 0)),
            pl.BlockSpec((_TRASH, _D), lambda i: (0, 0)),
        ],
        out_specs=pl.BlockSpec((blk, _D), lambda i: (i, 0)),
        out_shape=jax.ShapeDtypeStruct((_T, _D), jnp.float32),
    )(slot_g, g1, g2, nsf, xf, y)


def _wcast_body(w1_ref, w2_ref, s_ref, w1b_ref, w2b_ref):
    w1b_ref[...] = w1_ref[...].astype(jnp.bfloat16)
    w2b_ref[...] = w2_ref[...].astype(jnp.bfloat16)


def _wcast(W1, W2, dep):
    return pl.pallas_call(
        _wcast_body,
        grid=(_E, _HB),
        in_specs=[
            pl.BlockSpec((1, _D, _HBLK), lambda e, h: (e, 0, h)),
            pl.BlockSpec((1, _HBLK, _D), lambda e, h: (e, h, 0)),
            pl.BlockSpec((1, 1), lambda e, h: (0, 0)),
        ],
        out_specs=[
            pl.BlockSpec((1, _D, _HBLK), lambda e, h: (e, 0, h)),
            pl.BlockSpec((1, _HBLK, _D), lambda e, h: (e, h, 0)),
        ],
        out_shape=[
            jax.ShapeDtypeStruct((_E, _D, _H), jnp.bfloat16),
            jax.ShapeDtypeStruct((_E, _H, _D), jnp.bfloat16),
        ],
    )(W1, W2, dep)


def kernel(x, noise, Wr, br, Wn, bn, Ws, bs, W1, b1, W2, b2):
    B, S, D = x.shape
    xf = x.reshape(_T, _D)
    nf = noise.reshape(_T, _E)
    wcat = jnp.concatenate([Wr, Wn, Ws], axis=1)
    bcat = jnp.concatenate([br, bn, bs]).reshape(1, 2 * _E + 1)

    slot_s, slot_g, g1o, g2o, nsf = _router(xf, nf, wcat, bcat)
    sS = jnp.concatenate([slot_s[:, 0], slot_s[:, 1]])
    dep = nsf[0:1, 0:1]
    xg = _dispatch(xf, sS)
    W1b, W2b = _wcast(W1, W2, dep)
    y = _mlp(xg, W1b, b1, W2b, b2)
    final = _combine(slot_g, g1o, g2o, nsf, xf, y)
    return final.reshape(B, S, D)


# speedup vs baseline: 1.2027x; 1.2027x over previous
import functools

import jax
import jax.numpy as jnp
from jax import lax
from jax.experimental import pallas as pl
from jax.experimental.pallas import tpu as pltpu
from jax.experimental.pallas import tpu_sc as plsc

_T = 2048
_D = 768
_E = 8
_H = 3072
_CAP = 512
_TRASH = _E * _CAP
_XG_ROWS = _TRASH + 8
_NW = 32
_CHUNK = (2 * _T) // _NW
_NEG_INF = float("-inf")


def _router_body(xf_ref, nf_ref, w_ref, b_ref,
                 slot_s_ref, slot_g_ref, g1_ref, g2_ref, ns_ref):
    r = jnp.dot(xf_ref[...].astype(jnp.bfloat16),
                w_ref[...].astype(jnp.bfloat16),
                preferred_element_type=jnp.float32) + b_ref[...]
    logits = r[:, 0:_E]
    nlog = r[:, _E:2 * _E]
    z = r[:, 2 * _E:2 * _E + 1]
    sp = jnp.maximum(nlog, 0.0) + jnp.log1p(jnp.exp(-jnp.abs(nlog)))
    noisy = logits + nf_ref[...] * sp

    iota8 = lax.broadcasted_iota(jnp.int32, (_T, _E), 1)
    v1 = jnp.max(noisy, axis=1, keepdims=True)
    i1 = jnp.min(jnp.where(noisy == v1, iota8, _E), axis=1, keepdims=True)
    oh1 = iota8 == i1
    noisy2 = jnp.where(oh1, _NEG_INF, noisy)
    v2 = jnp.max(noisy2, axis=1, keepdims=True)
    i2 = jnp.min(jnp.where(noisy2 == v2, iota8, _E), axis=1, keepdims=True)
    oh2 = iota8 == i2

    ed = jnp.exp(v2 - v1)
    g1 = 1.0 / (1.0 + ed)
    g2 = ed / (1.0 + ed)

    ns = z <= 0.0
    nsf = ns.astype(jnp.float32)

    mask = jnp.where(jnp.logical_and(jnp.logical_or(oh1, oh2), ns), 1.0, 0.0)
    cum = mask
    sh = 1
    while sh < _T:
        cum = cum + jnp.concatenate(
            [jnp.zeros((sh, _E), jnp.float32), cum[:_T - sh]], axis=0)
        sh *= 2
    rank = cum - mask

    num_ns = jnp.sum(nsf, axis=0, keepdims=True)
    cap = jnp.floor(num_ns * 0.25)

    rank1 = jnp.sum(jnp.where(oh1, rank, 0.0), axis=1, keepdims=True)
    rank2 = jnp.sum(jnp.where(oh2, rank, 0.0), axis=1, keepdims=True)
    val1 = jnp.logical_and(ns, rank1 < cap)
    val2 = jnp.logical_and(ns, rank2 < cap)

    slot1 = i1 * _CAP + rank1.astype(jnp.int32)
    slot2 = i2 * _CAP + rank2.astype(jnp.int32)
    slot_s_ref[...] = jnp.concatenate(
        [jnp.where(val1, slot1, _TRASH), jnp.where(val2, slot2, _TRASH)], axis=1)
    slot_g_ref[...] = jnp.concatenate(
        [jnp.where(val1, slot1, 0), jnp.where(val2, slot2, 0)], axis=1)
    g1_ref[...] = g1 * val1.astype(jnp.float32)
    g2_ref[...] = g2 * val2.astype(jnp.float32)
    ns_ref[...] = nsf


def _router(xf, nf, wcat, bcat):
    return pl.pallas_call(
        _router_body,
        out_shape=[
            jax.ShapeDtypeStruct((_T, 2), jnp.int32),
            jax.ShapeDtypeStruct((_T, 2), jnp.int32),
            jax.ShapeDtypeStruct((_T, 1), jnp.float32),
            jax.ShapeDtypeStruct((_T, 1), jnp.float32),
            jax.ShapeDtypeStruct((_T, 1), jnp.float32),
        ],
    )(xf, nf, wcat, bcat)


_NSTR = 4
_SUB = _CHUNK // _NSTR


def _dispatch(xf, idx):
    @functools.partial(
        pl.kernel,
        out_type=jax.ShapeDtypeStruct((_XG_ROWS, _D), jnp.float32),
        mesh=plsc.VectorSubcoreMesh(core_axis_name="c", subcore_axis_name="s"),
        scratch_types=(
            [pltpu.VMEM((_SUB,), jnp.int32) for _ in range(_NSTR)]
            + [pltpu.VMEM((_CHUNK, _D), jnp.float32)]
            + [pltpu.SemaphoreType.DMA for _ in range(_NSTR + 1)]
        ),
    )
    def k(xf_hbm, idx_hbm, xg_hbm, i0, i1, i2, i3, rows_v, s0, s1, s2, s3, sr):
        wid = lax.axis_index("s") * 2 + lax.axis_index("c")
        base = wid * _CHUNK
        tok = lax.rem(base, _T)
        rcp = pltpu.make_async_copy(xf_hbm.at[pl.ds(tok, _CHUNK)], rows_v, sr)
        rcp.start()
        ivs = [i0, i1, i2, i3]
        sems = [s0, s1, s2, s3]
        for c in range(_NSTR):
            pltpu.sync_copy(idx_hbm.at[pl.ds(base + c * _SUB, _SUB)], ivs[c])
        rcp.wait()
        cps = []
        for c in range(_NSTR):
            cp = pltpu.make_async_copy(
                rows_v.at[pl.ds(c * _SUB, _SUB)], xg_hbm.at[ivs[c]], sems[c])
            cp.start()
            cps.append(cp)
        for cp in cps:
            cp.wait()

    return k(xf, idx)


def _mlp_body(xg_ref, w1_ref, b1_ref, w2_ref, b2_ref, y_ref, acc_ref):
    h_idx = pl.program_id(1)
    xraw = xg_ref[...]
    xb = jnp.where(jnp.abs(xraw) < 1e30, xraw, 0.0).astype(jnp.bfloat16)
    w1 = w1_ref[0].astype(jnp.bfloat16)
    h = jnp.dot(xb, w1, preferred_element_type=jnp.float32) + b1_ref[0]
    hb = jnp.maximum(h, 0.0).astype(jnp.bfloat16)
    w2 = w2_ref[0].astype(jnp.bfloat16)
    part = jnp.dot(hb, w2, preferred_element_type=jnp.float32)

    @pl.when(h_idx == 0)
    def _():
        acc_ref[...] = part + b2_ref[0]

    @pl.when(h_idx != 0)
    def _():
        y_ref[...] = (acc_ref[...] + part).astype(jnp.bfloat16)


_HB = 2
_HBLK = _H // _HB


def _mlp(xg, W1, b1, W2, b2):
    return pl.pallas_call(
        _mlp_body,
        grid=(_E, _HB),
        in_specs=[
            pl.BlockSpec((_CAP, _D), lambda e, h: (e, 0)),
            pl.BlockSpec((1, _D, _HBLK), lambda e, h: (e, 0, h)),
            pl.BlockSpec((1, 1, _HBLK), lambda e, h: (e, 0, h)),
            pl.BlockSpec((1, _HBLK, _D), lambda e, h: (e, h, 0)),
            pl.BlockSpec((1, 1, _D), lambda e, h: (e, 0, 0)),
        ],
        out_specs=pl.BlockSpec((_CAP, _D), lambda e, h: (e, 0)),
        out_shape=jax.ShapeDtypeStruct((_E * _CAP, _D), jnp.bfloat16),
        scratch_shapes=[pltpu.VMEM((_CAP, _D), jnp.float32)],
    )(xg, W1, b1.reshape(_E, 1, _H), W2, b2.reshape(_E, 1, _D))


_CB = 4


def _combine_body(sg_ref, g1_ref, g2_ref, ns_ref, xf_ref, y_ref, out_ref):
    s1 = sg_ref[...][:, 0:1]
    s2 = sg_ref[...][:, 1:2]
    iota = lax.broadcasted_iota(jnp.int32, (_T // _CB, _TRASH), 1)
    c = (jnp.where(iota == s1, g1_ref[...], 0.0)
         + jnp.where(iota == s2, g2_ref[...], 0.0)).astype(jnp.bfloat16)
    upd = jnp.dot(c, y_ref[...], preferred_element_type=jnp.float32)
    out_ref[...] = jnp.where(ns_ref[...] > 0.0, upd, xf_ref[...])


def _combine(slot_g, g1, g2, nsf, xf, y):
    blk = _T // _CB
    return pl.pallas_call(
        _combine_body,
        grid=(_CB,),
        in_specs=[
            pl.BlockSpec((blk, 2), lambda i: (i, 0)),
            pl.BlockSpec((blk, 1), lambda i: (i, 0)),
            pl.BlockSpec((blk, 1), lambda i: (i, 0)),
            pl.BlockSpec((blk, 1), lambda i: (i, 0)),
            pl.BlockSpec((blk, _D), lambda i: (i, 0)),
            pl.BlockSpec((_TRASH, _D), lambda i: (0, 0)),
        ],
        out_specs=pl.BlockSpec((blk, _D), lambda i: (i, 0)),
        out_shape=jax.ShapeDtypeStruct((_T, _D), jnp.float32),
    )(slot_g, g1, g2, nsf, xf, y)


def kernel(x, noise, Wr, br, Wn, bn, Ws, bs, W1, b1, W2, b2):
    B, S, D = x.shape
    xf = x.reshape(_T, _D)
    nf = noise.reshape(_T, _E)
    wcat = jnp.concatenate([Wr, Wn, Ws], axis=1)
    bcat = jnp.concatenate([br, bn, bs]).reshape(1, 2 * _E + 1)

    slot_s, slot_g, g1o, g2o, nsf = _router(xf, nf, wcat, bcat)
    sS = jnp.concatenate([slot_s[:, 0], slot_s[:, 1]])
    xg = _dispatch(xf, sS)
    y = _mlp(xg, W1, b1, W2, b2)
    final = _combine(slot_g, g1o, g2o, nsf, xf, y)
    return final.reshape(B, S, D)
